# Initial kernel scaffold; baseline (speedup 1.0000x reference)
import functools

import numpy as np
import jax
import jax.numpy as jnp
from jax.experimental import pallas as pl

_SEASON_LENGTH = 24
_LOWESS_FRAC = 0.6
_TOP_K_FRAC = 0.05
_PADP = 128


@functools.lru_cache(maxsize=None)
def _consts(L):
    x = np.arange(L, dtype=np.float64)
    r = max(2, int(_LOWESS_FRAC * L))
    dist = np.abs(x[:, None] - x[None, :])
    h = np.sort(dist, axis=1)[:, r - 1]
    u = dist / np.maximum(h[:, None], 1e-12)
    uc = np.clip(1.0 - u * u * u, 0.0, 1.0)
    w = (uc * uc * uc)
    W0 = w.T
    W1 = (w * x[None, :]).T
    W2 = (w * (x * x)[None, :]).T
    W012 = np.concatenate([W0, W1, W2], axis=1).astype(np.float32)
    csums = W012.sum(axis=0, keepdims=True).astype(np.float32)

    period = min(_SEASON_LENGTH, L)
    phase = np.arange(L) % period
    oh = np.zeros((L, _PADP), np.float32)
    oh[np.arange(L), phase] = 1.0
    counts = oh.sum(axis=0, keepdims=True)
    counts[counts == 0.0] = 1.0
    oht = np.ascontiguousarray(oh.T)
    return W012, csums, oh, oht, counts.astype(np.float32)


def _kth_smallest_keys(keys, k, signed):
    S, L = keys.shape
    kk = jnp.full((S, 1), k, jnp.int32)
    prefix = jnp.zeros((S, 1), jnp.int32)
    if signed:
        neg = keys < 0
        cnt = jnp.sum(neg.astype(jnp.int32), axis=1, keepdims=True)
        in_low = kk < cnt
        prefix = jnp.where(in_low, jnp.int32(-2**31), jnp.int32(0))
        kk = jnp.where(in_low, kk, kk - cnt)
        cand = neg == in_low
    else:
        cand = jnp.ones(keys.shape, jnp.bool_)
    for b in range(30, -1, -1):
        bit = jnp.int32(1 << b)
        low = (keys & bit) == 0
        c2 = cand & low
        cnt = jnp.sum(c2.astype(jnp.int32), axis=1, keepdims=True)
        go_hi = kk >= cnt
        prefix = jnp.where(go_hi, prefix | bit, prefix)
        kk = jnp.where(go_hi, kk - cnt, kk)
        if b > 0:
            cand = jnp.where(go_hi, cand & jnp.logical_not(low), c2)
    return prefix


def _key_from_f32(x):
    i = jax.lax.bitcast_convert_type(x, jnp.int32)
    return i ^ (jax.lax.shift_right_arithmetic(i, 31) & jnp.int32(0x7FFFFFFF))


def _f32_from_key(kv):
    i = kv ^ (jax.lax.shift_right_arithmetic(kv, 31) & jnp.int32(0x7FFFFFFF))
    return jax.lax.bitcast_convert_type(i, jnp.float32)


def _median_mean(absvals, L):
    keys = jax.lax.bitcast_convert_type(absvals, jnp.int32)
    klo = _kth_smallest_keys(keys, (L - 1) // 2, False)
    cnt_le = jnp.sum((keys <= klo).astype(jnp.int32), axis=1, keepdims=True)
    imax = jnp.int32(2**31 - 1)
    nxt = jnp.min(jnp.where(keys > klo, keys, imax), axis=1, keepdims=True)
    khi = jnp.where(cnt_le >= (L // 2 + 1), klo, nxt)
    lo = jax.lax.bitcast_convert_type(klo, jnp.float32)
    hi = jax.lax.bitcast_convert_type(khi, jnp.float32)
    return 0.5 * (lo + hi)


def _star_kernel(y_ref, w_ref, cs_ref, oh_ref, oht_ref, cnt_ref,
                 trend_ref, seasonal_ref, anom_ref, clean_ref,
                 maskf_ref, signed_ref, abs_ref, cutoff_ref):
    S, L = y_ref.shape
    f32 = jnp.float32
    y = y_ref[:]
    xt = jax.lax.broadcasted_iota(f32, (1, L), 1)

    def dot(a, b):
        return jax.lax.dot_general(a, b, (((1,), (0,)), ((), ())),
                                   preferred_element_type=f32)

    cs = cs_ref[:]
    sw, swx, swxx = cs[:, :L], cs[:, L:2 * L], cs[:, 2 * L:]
    qw = dot(y, w_ref[:, :2 * L])
    swy, swxy = qw[:, :L], qw[:, L:]
    yhat = None
    for it in range(4):
        denom = sw * swxx - swx * swx
        denom = jnp.where(jnp.abs(denom) < 1e-8, f32(1e-8), denom)
        b_ = (sw * swxy - swx * swy) / denom
        a_ = (swy - b_ * swx) / jnp.maximum(sw, 1e-12)
        yhat = a_ + b_ * xt
        if it == 3:
            break
        e = y - yhat
        s = _median_mean(jnp.abs(e), L)
        uu = e / jnp.maximum(6.0 * s, 1e-12)
        rho = jnp.clip(1.0 - uu * uu, 0.0, 1.0)
        rho = rho * rho
        pw = dot(rho, w_ref[:])
        sw, swx, swxx = pw[:, :L], pw[:, L:2 * L], pw[:, 2 * L:]
        qw = dot(rho * y, w_ref[:, :2 * L])
        swy, swxy = qw[:, :L], qw[:, L:]

    y0 = y[:, :1]
    bad = jnp.abs(y - y0) > (1e-8 + 1e-5 * jnp.abs(y0))
    nbad = jnp.sum(bad.astype(jnp.int32), axis=1, keepdims=True)
    trend = jnp.where(nbad == 0, y, yhat)

    den_t = jnp.where(jnp.abs(trend) < 1e-4, f32(1e-4), trend)
    detr = y / den_t
    sums = dot(detr, oh_ref[:])
    means = sums / cnt_ref[:]
    seasonal = dot(means, oht_ref[:])
    den_s = jnp.where(jnp.abs(seasonal) < 1e-4, f32(1e-4), seasonal)
    resid = detr / den_s

    rkeys = _key_from_f32(resid)
    center = _f32_from_key(_kth_smallest_keys(rkeys, (L - 1) // 2, True))
    dev = jnp.abs(resid - center)
    dkeys = jax.lax.bitcast_convert_type(dev, jnp.int32)
    mad_raw = jax.lax.bitcast_convert_type(
        _kth_smallest_keys(dkeys, (L - 1) // 2, False), f32)
    mad = jnp.maximum(mad_raw, 1e-4)
    signed = 0.6745 * (resid - center) / mad
    absS = jnp.abs(signed)
    ktop = max(1, int(np.ceil(_TOP_K_FRAC * L)))
    akeys = jax.lax.bitcast_convert_type(absS, jnp.int32)
    cutoff = jax.lax.bitcast_convert_type(
        _kth_smallest_keys(akeys, L - ktop, False), f32)
    mask = absS >= cutoff

    trend_ref[:] = trend
    seasonal_ref[:] = seasonal
    anom_ref[:] = jnp.where(mask, resid, f32(1.0))
    clean_ref[:] = jnp.where(mask, f32(1.0), resid)
    maskf_ref[:] = mask.astype(f32)
    signed_ref[:] = signed
    abs_ref[:] = absS
    cutoff_ref[:] = cutoff


def kernel(insample_y):
    B, L, C = insample_y.shape
    S = B * C
    W012, csums, oh, oht, counts = _consts(L)
    y_sl = insample_y.transpose(0, 2, 1).reshape(S, L)
    fS = jax.ShapeDtypeStruct((S, L), jnp.float32)
    outs = pl.pallas_call(
        _star_kernel,
        out_shape=[fS, fS, fS, fS, fS, fS, fS,
                   jax.ShapeDtypeStruct((S, 1), jnp.float32)],
    )(y_sl, jnp.asarray(W012), jnp.asarray(csums), jnp.asarray(oh),
      jnp.asarray(oht), jnp.asarray(counts))

    def back(a):
        return a.reshape(B, C, L).transpose(0, 2, 1)

    trend, seasonal, anomalies, cleaned, maskf, signed, absS = map(
        back, outs[:7])
    mask = maskf > 0
    cutoff = outs[7].reshape(B, C, 1).transpose(0, 2, 1)
    return (trend, seasonal, anomalies, cleaned, mask,
            signed, absS, absS, cutoff)


# speedup vs baseline: 2.6495x; 2.6495x over previous
import numpy as np
import jax
import jax.numpy as jnp
from jax.experimental import pallas as pl

_SEASON_LENGTH = 24
_LOWESS_FRAC = 0.6
_TOP_K_FRAC = 0.05

_IMAX = 2**31 - 1


def _kth_smallest_keys(keys, k, signed):
    S, L = keys.shape
    kk = jnp.full((S, 1), k, jnp.int32)
    prefix = jnp.zeros((S, 1), jnp.int32)
    if signed:
        negbit = jax.lax.shift_right_logical(keys, 31)
        cnt = jnp.sum(negbit, axis=1, keepdims=True)
        in_low = kk < cnt
        prefix = jnp.where(in_low, jnp.int32(-2**31), jnp.int32(0))
        kk = jnp.where(in_low, kk, kk - cnt)
        cand = jnp.where(in_low, negbit, 1 - negbit)
    else:
        cand = jnp.ones(keys.shape, jnp.int32)
    for b in range(30, -1, -1):
        bit = jnp.int32(1 << b)
        lowbit = jax.lax.shift_right_logical(keys, b) & 1
        c2 = cand & (lowbit ^ 1)
        cnt = jnp.sum(c2, axis=1, keepdims=True)
        go_hi = kk >= cnt
        prefix = jnp.where(go_hi, prefix | bit, prefix)
        kk = jnp.where(go_hi, kk - cnt, kk)
        if b > 0:
            cand = jnp.where(go_hi, cand - c2, c2)
    return prefix


def _median_even(absvals):
    keys = jax.lax.bitcast_convert_type(absvals, jnp.int32)
    S, n = keys.shape
    klo = _kth_smallest_keys(keys, (n - 1) // 2, False)
    ge = 1 - (jax.lax.shift_right_logical(klo - keys, 31) & 1)
    cnt_le = jnp.sum(ge, axis=1, keepdims=True)
    gt = jax.lax.shift_right_logical(klo - keys, 31) & 1
    vals = jnp.int32(_IMAX) + (keys - jnp.int32(_IMAX)) * gt
    nxt = jnp.min(vals, axis=1, keepdims=True)
    khi = jnp.where(cnt_le >= (n // 2 + 1), klo, nxt)
    lo = jax.lax.bitcast_convert_type(klo, jnp.float32)
    hi = jax.lax.bitcast_convert_type(khi, jnp.float32)
    return (lo + hi) * 0.5


def _key_from_f32(x):
    i = jax.lax.bitcast_convert_type(x, jnp.int32)
    return i ^ (jax.lax.shift_right_arithmetic(i, 31) & jnp.int32(_IMAX))


def _f32_from_key(kv):
    i = kv ^ (jax.lax.shift_right_arithmetic(kv, 31) & jnp.int32(_IMAX))
    return jax.lax.bitcast_convert_type(i, jnp.float32)


def _star_kernel(y_ref, wt_ref, sw1_ref, oh_ref, cnt_ref,
                 trend_ref, seasonal_ref, anom_ref, clean_ref,
                 maskf_ref, signed_ref, abs_ref, cutoff_ref):
    S, L = y_ref.shape
    f32 = jnp.float32
    y = y_ref[:]
    wt = wt_ref[:]

    def dot(a, b):
        return jax.lax.dot_general(a, b, (((1,), (0,)), ((), ())),
                                   preferred_element_type=f32)

    x_row = jax.lax.broadcasted_iota(jnp.int32, (1, L), 1).astype(f32)
    x2_row = x_row * x_row
    rho = jnp.ones_like(y)
    yhat = y
    for it in range(4):
        sw = sw1_ref[:] if it == 0 else dot(rho, wt)
        swx = dot(rho * x_row, wt)
        swy = dot(rho * y, wt)
        swxx = dot(rho * x2_row, wt)
        swxy = dot(rho * x_row * y, wt)
        denom = sw * swxx - swx * swx
        denom = jnp.where(jnp.abs(denom) < 1e-8, f32(1e-8), denom)
        b_ = (sw * swxy - swx * swy) / denom
        a_ = (swy - b_ * swx) / jnp.maximum(sw, 1e-12)
        yhat = a_ + b_ * x_row
        if it < 3:
            e = y - yhat
            s = _median_even(jnp.abs(e))
            uu = e / jnp.maximum(6.0 * s, 1e-12)
            rho = jnp.clip(1.0 - uu * uu, 0.0, 1.0) ** 2

    y0 = y[:, :1]
    bad = jnp.abs(y - y0) > (1e-8 + 1e-5 * jnp.abs(y0))
    nbad = jnp.sum(jnp.where(bad, 1, 0), axis=1, keepdims=True)
    trend = jnp.where(nbad == 0, y, yhat)

    den_t = jnp.where(jnp.abs(trend) < 1e-4, f32(1e-4), trend)
    detr = y / den_t
    period = oh_ref.shape[1]
    sums = dot(detr, oh_ref[:])
    means = sums / cnt_ref[:]
    phase_row = jax.lax.broadcasted_iota(jnp.int32, (1, L), 1) % period
    seasonal = jnp.zeros_like(y)
    for p in range(period):
        seasonal = jnp.where(phase_row == p,
                             jax.lax.broadcast_in_dim(means[:, p:p+1],
                                                      (S, L), (0, 1)),
                             seasonal)
    den_s = jnp.where(jnp.abs(seasonal) < 1e-4, f32(1e-4), seasonal)
    resid = detr / den_s

    rkeys = _key_from_f32(resid)
    center = _f32_from_key(_kth_smallest_keys(rkeys, (L - 1) // 2, True))
    dev = jnp.abs(resid - center)
    dkeys = jax.lax.bitcast_convert_type(dev, jnp.int32)
    mad_raw = jax.lax.bitcast_convert_type(
        _kth_smallest_keys(dkeys, (L - 1) // 2, False), f32)
    mad = jnp.maximum(mad_raw, 1e-4)
    signed = 0.6745 * (resid - center) / mad
    absS = jnp.abs(signed)
    ktop = max(1, int(np.ceil(_TOP_K_FRAC * L)))
    akeys = jax.lax.bitcast_convert_type(absS, jnp.int32)
    cutoff = jax.lax.bitcast_convert_type(
        _kth_smallest_keys(akeys, L - ktop, False), f32)
    mask = absS >= cutoff

    trend_ref[:] = trend
    seasonal_ref[:] = seasonal
    anom_ref[:] = jnp.where(mask, resid, f32(1.0))
    clean_ref[:] = jnp.where(mask, f32(1.0), resid)
    maskf_ref[:] = jnp.where(mask, f32(1.0), f32(0.0))
    signed_ref[:] = signed
    abs_ref[:] = absS
    cutoff_ref[:] = cutoff


def kernel(insample_y):
    B, L, C = insample_y.shape
    S = B * C
    dt = insample_y.dtype

    x = jnp.arange(L, dtype=dt)
    r = max(2, int(_LOWESS_FRAC * L))
    dist = jnp.abs(x[:, None] - x[None, :])
    h = jnp.sort(dist, axis=1)[:, r - 1]
    u = dist / jnp.maximum(h[:, None], 1e-12)
    w = jnp.clip(1.0 - u ** 3, 0.0, 1.0) ** 3
    wt = w.T
    sw1 = jnp.sum(wt, axis=0, keepdims=True)

    period = min(_SEASON_LENGTH, L)
    phase = jnp.arange(L) % period
    onehot = jax.nn.one_hot(phase, period, dtype=dt)
    counts = onehot.sum(axis=0)[None, :]

    y_sl = insample_y.transpose(0, 2, 1).reshape(S, L)
    fS = jax.ShapeDtypeStruct((S, L), jnp.float32)
    outs = pl.pallas_call(
        _star_kernel,
        out_shape=[fS, fS, fS, fS, fS, fS, fS,
                   jax.ShapeDtypeStruct((S, 1), jnp.float32)],
    )(y_sl, wt, sw1, onehot, counts)

    def back(a):
        return a.reshape(B, C, L).transpose(0, 2, 1)

    trend, seasonal, anomalies, cleaned, maskf, signed, absS = map(
        back, outs[:7])
    mask = maskf > 0
    cutoff = outs[7].reshape(B, C, 1).transpose(0, 2, 1)
    return (trend, seasonal, anomalies, cleaned, mask,
            signed, absS, absS, cutoff)


# speedup vs baseline: 2.8017x; 1.0574x over previous
import numpy as np
import jax
import jax.numpy as jnp
from jax.experimental import pallas as pl

_SEASON_LENGTH = 24
_LOWESS_FRAC = 0.6
_TOP_K_FRAC = 0.05

_IMAX = 2**31 - 1


def _kth_smallest_keys(keys, k, signed):
    S, L = keys.shape
    kk = jnp.full((S, 1), k, jnp.int32)
    prefix = jnp.zeros((S, 1), jnp.int32)
    if signed:
        negbit = jax.lax.shift_right_logical(keys, 31)
        cnt = jnp.sum(negbit, axis=1, keepdims=True)
        in_low = kk < cnt
        prefix = jnp.where(in_low, jnp.int32(-2**31), jnp.int32(0))
        kk = jnp.where(in_low, kk, kk - cnt)
        cand = jnp.where(in_low, negbit, 1 - negbit)
    else:
        cand = jnp.ones(keys.shape, jnp.int32)
    for b in range(30, -1, -1):
        bit = jnp.int32(1 << b)
        shifted = jax.lax.shift_left(keys, 31 - b)
        c2 = jnp.where(shifted >= 0, cand, 0)
        cnt = jnp.sum(c2, axis=1, keepdims=True)
        go_hi = kk >= cnt
        prefix = jnp.where(go_hi, prefix | bit, prefix)
        kk = jnp.where(go_hi, kk - cnt, kk)
        if b > 0:
            cand = jnp.where(go_hi, cand - c2, c2)
    return prefix


def _median_even(absvals):
    keys = jax.lax.bitcast_convert_type(absvals, jnp.int32)
    S, n = keys.shape
    klo = _kth_smallest_keys(keys, (n - 1) // 2, False)
    ge = 1 - (jax.lax.shift_right_logical(klo - keys, 31) & 1)
    cnt_le = jnp.sum(ge, axis=1, keepdims=True)
    gt = jax.lax.shift_right_logical(klo - keys, 31) & 1
    vals = jnp.int32(_IMAX) + (keys - jnp.int32(_IMAX)) * gt
    nxt = jnp.min(vals, axis=1, keepdims=True)
    khi = jnp.where(cnt_le >= (n // 2 + 1), klo, nxt)
    lo = jax.lax.bitcast_convert_type(klo, jnp.float32)
    hi = jax.lax.bitcast_convert_type(khi, jnp.float32)
    return (lo + hi) * 0.5


def _key_from_f32(x):
    i = jax.lax.bitcast_convert_type(x, jnp.int32)
    return i ^ (jax.lax.shift_right_arithmetic(i, 31) & jnp.int32(_IMAX))


def _f32_from_key(kv):
    i = kv ^ (jax.lax.shift_right_arithmetic(kv, 31) & jnp.int32(_IMAX))
    return jax.lax.bitcast_convert_type(i, jnp.float32)


def _star_kernel(y_ref, wt_ref, sw1_ref, oh_ref, cnt_ref,
                 trend_ref, seasonal_ref, anom_ref, clean_ref,
                 maskf_ref, signed_ref, abs_ref, cutoff_ref):
    S, L = y_ref.shape
    f32 = jnp.float32
    y = y_ref[:]
    wt = wt_ref[:]

    def dot(a, b):
        return jax.lax.dot_general(a, b, (((1,), (0,)), ((), ())),
                                   preferred_element_type=f32)

    x_row = jax.lax.broadcasted_iota(jnp.int32, (1, L), 1).astype(f32)
    x2_row = x_row * x_row
    rho = jnp.ones_like(y)
    yhat = y
    for it in range(4):
        sw = sw1_ref[:] if it == 0 else dot(rho, wt)
        swx = dot(rho * x_row, wt)
        swy = dot(rho * y, wt)
        swxx = dot(rho * x2_row, wt)
        swxy = dot(rho * x_row * y, wt)
        denom = sw * swxx - swx * swx
        denom = jnp.where(jnp.abs(denom) < 1e-8, f32(1e-8), denom)
        b_ = (sw * swxy - swx * swy) / denom
        a_ = (swy - b_ * swx) / jnp.maximum(sw, 1e-12)
        yhat = a_ + b_ * x_row
        if it < 3:
            e = y - yhat
            s = _median_even(jnp.abs(e))
            uu = e / jnp.maximum(6.0 * s, 1e-12)
            rho = jnp.clip(1.0 - uu * uu, 0.0, 1.0) ** 2

    y0 = y[:, :1]
    bad = jnp.abs(y - y0) > (1e-8 + 1e-5 * jnp.abs(y0))
    nbad = jnp.sum(jnp.where(bad, 1, 0), axis=1, keepdims=True)
    trend = jnp.where(nbad == 0, y, yhat)

    den_t = jnp.where(jnp.abs(trend) < 1e-4, f32(1e-4), trend)
    detr = y / den_t
    period = oh_ref.shape[1]
    sums = dot(detr, oh_ref[:])
    means = sums / cnt_ref[:]
    phase_row = jax.lax.broadcasted_iota(jnp.int32, (1, L), 1) % period
    seasonal = jnp.zeros_like(y)
    for p in range(period):
        seasonal = jnp.where(phase_row == p,
                             jax.lax.broadcast_in_dim(means[:, p:p+1],
                                                      (S, L), (0, 1)),
                             seasonal)
    den_s = jnp.where(jnp.abs(seasonal) < 1e-4, f32(1e-4), seasonal)
    resid = detr / den_s

    rkeys = _key_from_f32(resid)
    center = _f32_from_key(_kth_smallest_keys(rkeys, (L - 1) // 2, True))
    dev = jnp.abs(resid - center)
    dkeys = jax.lax.bitcast_convert_type(dev, jnp.int32)
    mad_raw = jax.lax.bitcast_convert_type(
        _kth_smallest_keys(dkeys, (L - 1) // 2, False), f32)
    mad = jnp.maximum(mad_raw, 1e-4)
    signed = 0.6745 * (resid - center) / mad
    absS = jnp.abs(signed)
    ktop = max(1, int(np.ceil(_TOP_K_FRAC * L)))
    akeys = jax.lax.bitcast_convert_type(absS, jnp.int32)
    cutoff = jax.lax.bitcast_convert_type(
        _kth_smallest_keys(akeys, L - ktop, False), f32)
    mask = absS >= cutoff

    trend_ref[:] = trend
    seasonal_ref[:] = seasonal
    anom_ref[:] = jnp.where(mask, resid, f32(1.0))
    clean_ref[:] = jnp.where(mask, f32(1.0), resid)
    maskf_ref[:] = jnp.where(mask, f32(1.0), f32(0.0))
    signed_ref[:] = signed
    abs_ref[:] = absS
    cutoff_ref[:] = cutoff


def kernel(insample_y):
    B, L, C = insample_y.shape
    S = B * C
    dt = insample_y.dtype

    x = jnp.arange(L, dtype=dt)
    r = max(2, int(_LOWESS_FRAC * L))
    dist = jnp.abs(x[:, None] - x[None, :])
    h = jnp.sort(dist, axis=1)[:, r - 1]
    u = dist / jnp.maximum(h[:, None], 1e-12)
    w = jnp.clip(1.0 - u ** 3, 0.0, 1.0) ** 3
    wt = w.T
    sw1 = jnp.sum(wt, axis=0, keepdims=True)

    period = min(_SEASON_LENGTH, L)
    phase = jnp.arange(L) % period
    onehot = jax.nn.one_hot(phase, period, dtype=dt)
    counts = onehot.sum(axis=0)[None, :]

    y_sl = insample_y.transpose(0, 2, 1).reshape(S, L)
    fS = jax.ShapeDtypeStruct((S, L), jnp.float32)
    outs = pl.pallas_call(
        _star_kernel,
        out_shape=[fS, fS, fS, fS, fS, fS, fS,
                   jax.ShapeDtypeStruct((S, 1), jnp.float32)],
    )(y_sl, wt, sw1, onehot, counts)

    def back(a):
        return a.reshape(B, C, L).transpose(0, 2, 1)

    trend, seasonal, anomalies, cleaned, maskf, signed, absS = map(
        back, outs[:7])
    mask = maskf > 0
    cutoff = outs[7].reshape(B, C, 1).transpose(0, 2, 1)
    return (trend, seasonal, anomalies, cleaned, mask,
            signed, absS, absS, cutoff)
